# Initial kernel scaffold; baseline (speedup 1.0000x reference)
import jax
import jax.numpy as jnp
from jax.experimental import pallas as pl


def kernel(x, edge_index, batch, W1, b1, W2, b2, Wr, br):
    raise NotImplementedError("write your pallas kernel here")


# speedup vs baseline: 25.1192x; 25.1192x over previous
import functools

import jax
import jax.numpy as jnp
from jax import lax
from jax.experimental import pallas as pl
from jax.experimental.pallas import tpu as pltpu
from jax.experimental.pallas import tpu_sc as plsc

N = 10000
E = 320000
DIN = 128
DH = 32
DOUT = 32
G = 128

NC = 2
NS = 16
NW = NC * NS
C = 128
K = -(-E // (NW * C))
EP = NW * K * C
NPAD = 10240
NT = NPAD // NS
CW = 16

_sc_mesh = plsc.VectorSubcoreMesh(
    core_axis_name="c", subcore_axis_name="s", num_cores=NC, num_subcores=NS
)


@functools.partial(
    pl.kernel,
    out_type=jax.ShapeDtypeStruct((NC, NPAD, CW), jnp.float32),
    mesh=_sc_mesh,
    scratch_types=[
        pltpu.VMEM_SHARED((NPAD, CW), jnp.float32),
        pltpu.VMEM((NT, CW), jnp.float32),
        pltpu.VMEM((K, C), jnp.int32),
        pltpu.VMEM((C, CW), jnp.float32),
    ],
    compiler_params=pltpu.CompilerParams(use_tc_tiling_on_sc=False),
)
def _sc_count(dst_hbm, ones_hbm, zero_hbm, out_hbm, accum_sh, stage_v, idx_v,
              ones_v):
    cid = lax.axis_index("c")
    sid = lax.axis_index("s")
    wid = sid * NC + cid
    rows = pl.ds(sid * NT, NT)
    pltpu.sync_copy(zero_hbm.at[rows], stage_v)
    pltpu.sync_copy(stage_v, accum_sh.at[rows])
    pltpu.sync_copy(ones_hbm, ones_v)
    pltpu.sync_copy(dst_hbm.at[wid], idx_v)
    plsc.subcore_barrier()

    def chunk(k, carry):
        pltpu.sync_copy(ones_v, accum_sh.at[idx_v.at[k]], add=True)
        return carry

    lax.fori_loop(0, K, chunk, 0)
    plsc.subcore_barrier()
    pltpu.sync_copy(accum_sh.at[rows], stage_v)
    pltpu.sync_copy(stage_v, out_hbm.at[cid, rows])


@functools.partial(
    pl.kernel,
    out_type=jax.ShapeDtypeStruct((NC, NPAD, DH), jnp.float32),
    mesh=_sc_mesh,
    scratch_types=[
        pltpu.VMEM_SHARED((NPAD, DH), jnp.float32),
        pltpu.VMEM((NT, DH), jnp.float32),
        pltpu.VMEM((K, C), jnp.int32),
        pltpu.VMEM((K, C), jnp.int32),
        pltpu.VMEM((C, DH), jnp.float32),
        pltpu.SemaphoreType.DMA,
    ],
    compiler_params=pltpu.CompilerParams(use_tc_tiling_on_sc=False),
)
def _sc_layer(y_hbm, src_hbm, dst_hbm, zero_hbm, out_hbm, accum_sh,
              stage_v, sidx_v, didx_v, gbuf_v, sem):
    cid = lax.axis_index("c")
    sid = lax.axis_index("s")
    wid = sid * NC + cid
    rows = pl.ds(sid * NT, NT)
    pltpu.sync_copy(zero_hbm.at[rows], stage_v)
    pltpu.sync_copy(stage_v, accum_sh.at[rows])
    pltpu.sync_copy(src_hbm.at[wid], sidx_v)
    pltpu.sync_copy(dst_hbm.at[wid], didx_v)
    plsc.subcore_barrier()

    def chunk(k, carry):
        pltpu.async_copy(y_hbm.at[sidx_v.at[k]], gbuf_v, sem).wait()
        pltpu.sync_copy(gbuf_v, accum_sh.at[didx_v.at[k]], add=True)
        return carry

    lax.fori_loop(0, K, chunk, 0)
    plsc.subcore_barrier()
    pltpu.sync_copy(accum_sh.at[rows], stage_v)
    pltpu.sync_copy(stage_v, out_hbm.at[cid, rows])


def _tc_matmul1_body(x_ref, w_ref, out_ref):
    out_ref[...] = jnp.dot(x_ref[...], w_ref[...],
                           preferred_element_type=jnp.float32, precision=lax.Precision.HIGHEST)


def _tc_scale_body(cnt_ref, xw_ref, yp_ref, dinv_ref):
    cnt = cnt_ref[0, 0:N, 0:1] + cnt_ref[1, 0:N, 0:1]
    dinv = lax.rsqrt(cnt + 1.0)
    dinv_ref[...] = dinv
    yp_ref[...] = jnp.zeros((NPAD, DH), jnp.float32)
    yp_ref[0:N, :] = xw_ref[...] * dinv


def _tc_mid_body(p_ref, yp_ref, dinv_ref, b_ref, w_ref, out_ref):
    agg = p_ref[0, 0:N, :] + p_ref[1, 0:N, :] + yp_ref[0:N, :]
    h = jnp.maximum(dinv_ref[...] * agg + b_ref[...], 0.0)
    y2 = jnp.dot(h, w_ref[...], preferred_element_type=jnp.float32, precision=lax.Precision.HIGHEST)
    out_ref[...] = jnp.zeros((NPAD, DH), jnp.float32)
    out_ref[0:N, :] = y2 * dinv_ref[...]


def _tc_final_body(q_ref, yp_ref, dinv_ref, b_ref, batch_ref, wr_ref, br_ref,
                   out_ref):
    agg = q_ref[0, 0:N, :] + q_ref[1, 0:N, :] + yp_ref[0:N, :]
    h = jnp.maximum(dinv_ref[...] * agg + b_ref[...], 0.0)
    gids = lax.broadcasted_iota(jnp.int32, (G, N), 0)
    onehot_t = (batch_ref[...] == gids).astype(jnp.float32)
    pool = lax.dot_general(onehot_t, h, (((1,), (0,)), ((), ())),
                           preferred_element_type=jnp.float32, precision=lax.Precision.HIGHEST)
    cntg = lax.dot_general(onehot_t, jnp.ones((N, 1), jnp.float32),
                           (((1,), (0,)), ((), ())),
                           preferred_element_type=jnp.float32, precision=lax.Precision.HIGHEST)
    mean = pool / jnp.maximum(cntg, 1.0)
    out_ref[...] = jnp.dot(mean, wr_ref[...],
                           preferred_element_type=jnp.float32, precision=lax.Precision.HIGHEST) + br_ref[...]


_tc_matmul1 = pl.pallas_call(
    _tc_matmul1_body,
    out_shape=jax.ShapeDtypeStruct((N, DH), jnp.float32),
)

_tc_scale = pl.pallas_call(
    _tc_scale_body,
    out_shape=(
        jax.ShapeDtypeStruct((NPAD, DH), jnp.float32),
        jax.ShapeDtypeStruct((N, 1), jnp.float32),
    ),
)

_tc_mid = pl.pallas_call(
    _tc_mid_body,
    out_shape=jax.ShapeDtypeStruct((NPAD, DH), jnp.float32),
)

_tc_final = pl.pallas_call(
    _tc_final_body,
    out_shape=jax.ShapeDtypeStruct((G, DOUT), jnp.float32),
)


def kernel(x, edge_index, batch, W1, b1, W2, b2, Wr, br):
    src = edge_index[0]
    dst = edge_index[1]
    pad = jnp.full((EP - E,), N, jnp.int32)
    srcp = jnp.concatenate([src, pad]).reshape(NW, K, C)
    dstp = jnp.concatenate([dst, pad]).reshape(NW, K, C)
    zeros = jnp.zeros((NPAD, DH), jnp.float32)
    zeros_c = jnp.zeros((NPAD, CW), jnp.float32)
    ones_c = jnp.ones((C, CW), jnp.float32)
    batch_row = batch.reshape(1, N)
    b1r = b1.reshape(1, DH)
    b2r = b2.reshape(1, DH)
    brr = br.reshape(1, DOUT)

    cnt = _sc_count(dstp, ones_c, zeros_c)
    xw1 = _tc_matmul1(x, W1)
    y1p, dinv = _tc_scale(cnt, xw1)
    p = _sc_layer(y1p, srcp, dstp, zeros)
    y2p = _tc_mid(p, y1p, dinv, b1r, W2)
    q = _sc_layer(y2p, srcp, dstp, zeros)
    return _tc_final(q, y2p, dinv, b2r, batch_row, Wr, brr)


# speedup vs baseline: 28.1930x; 1.1224x over previous
import functools

import jax
import jax.numpy as jnp
from jax import lax
from jax.experimental import pallas as pl
from jax.experimental.pallas import tpu as pltpu
from jax.experimental.pallas import tpu_sc as plsc

N = 10000
E = 320000
DIN = 128
DH = 32
DOUT = 32
G = 128

NC = 2
NS = 16
NW = NC * NS
C = 128
K = -(-E // (NW * C))
K += K % 2
K2 = K // 2
EP = NW * K * C
NPAD = 10240
NT = NPAD // NS
CW = 16

_sc_mesh = plsc.VectorSubcoreMesh(
    core_axis_name="c", subcore_axis_name="s", num_cores=NC, num_subcores=NS
)


@functools.partial(
    pl.kernel,
    out_type=jax.ShapeDtypeStruct((NC, NPAD, CW), jnp.float32),
    mesh=_sc_mesh,
    scratch_types=[
        pltpu.VMEM_SHARED((NPAD, CW), jnp.float32),
        pltpu.VMEM((NT, CW), jnp.float32),
        pltpu.VMEM((K, C), jnp.int32),
        pltpu.VMEM((C, CW), jnp.float32),
    ],
    compiler_params=pltpu.CompilerParams(use_tc_tiling_on_sc=False),
)
def _sc_count(dst_hbm, ones_hbm, zero_hbm, out_hbm, accum_sh, stage_v, idx_v,
              ones_v):
    cid = lax.axis_index("c")
    sid = lax.axis_index("s")
    wid = sid * NC + cid
    rows = pl.ds(sid * NT, NT)
    pltpu.sync_copy(zero_hbm.at[rows], stage_v)
    pltpu.sync_copy(stage_v, accum_sh.at[rows])
    pltpu.sync_copy(ones_hbm, ones_v)
    pltpu.sync_copy(dst_hbm.at[wid], idx_v)
    plsc.subcore_barrier()

    def chunk(k, carry):
        pltpu.sync_copy(ones_v, accum_sh.at[idx_v.at[k]], add=True)
        return carry

    lax.fori_loop(0, K, chunk, 0)
    plsc.subcore_barrier()
    pltpu.sync_copy(accum_sh.at[rows], stage_v)
    pltpu.sync_copy(stage_v, out_hbm.at[cid, rows])


@functools.partial(
    pl.kernel,
    out_type=jax.ShapeDtypeStruct((NC, NPAD, DH), jnp.float32),
    mesh=_sc_mesh,
    scratch_types=[
        pltpu.VMEM_SHARED((NPAD, DH), jnp.float32),
        pltpu.VMEM((NT, DH), jnp.float32),
        pltpu.VMEM((K, C), jnp.int32),
        pltpu.VMEM((K, C), jnp.int32),
        pltpu.VMEM((C, DH), jnp.float32),
        pltpu.VMEM((C, DH), jnp.float32),
        pltpu.SemaphoreType.DMA,
        pltpu.SemaphoreType.DMA,
    ],
    compiler_params=pltpu.CompilerParams(use_tc_tiling_on_sc=False),
)
def _sc_layer(y_hbm, src_hbm, dst_hbm, zero_hbm, out_hbm, accum_sh,
              stage_v, sidx_v, didx_v, gbuf0_v, gbuf1_v, sem0, sem1):
    cid = lax.axis_index("c")
    sid = lax.axis_index("s")
    wid = sid * NC + cid
    rows = pl.ds(sid * NT, NT)
    pltpu.sync_copy(zero_hbm.at[rows], stage_v)
    pltpu.sync_copy(stage_v, accum_sh.at[rows])
    pltpu.sync_copy(src_hbm.at[wid], sidx_v)
    pltpu.sync_copy(dst_hbm.at[wid], didx_v)
    plsc.subcore_barrier()

    pltpu.async_copy(y_hbm.at[sidx_v.at[0]], gbuf0_v, sem0)

    def chunk_pair(i, carry):
        k0 = 2 * i
        pltpu.async_copy(y_hbm.at[sidx_v.at[k0 + 1]], gbuf1_v, sem1)
        pltpu.make_async_copy(y_hbm.at[sidx_v.at[k0]], gbuf0_v, sem0).wait()
        pltpu.sync_copy(gbuf0_v, accum_sh.at[didx_v.at[k0]], add=True)

        @pl.when(i + 1 < K2)
        def _():
            pltpu.async_copy(y_hbm.at[sidx_v.at[k0 + 2]], gbuf0_v, sem0)

        pltpu.make_async_copy(y_hbm.at[sidx_v.at[k0 + 1]], gbuf1_v,
                              sem1).wait()
        pltpu.sync_copy(gbuf1_v, accum_sh.at[didx_v.at[k0 + 1]], add=True)
        return carry

    lax.fori_loop(0, K2, chunk_pair, 0)
    plsc.subcore_barrier()
    pltpu.sync_copy(accum_sh.at[rows], stage_v)
    pltpu.sync_copy(stage_v, out_hbm.at[cid, rows])


def _tc_scale_body(cnt_ref, x_ref, w_ref, yp_ref, dinv_ref):
    cnt = cnt_ref[0, 0:N, 0:1] + cnt_ref[1, 0:N, 0:1]
    dinv = lax.rsqrt(cnt + 1.0)
    dinv_ref[...] = dinv
    xw = jnp.dot(x_ref[...], w_ref[...], preferred_element_type=jnp.float32,
                 precision=lax.Precision.HIGHEST)
    yp_ref[...] = jnp.zeros((NPAD, DH), jnp.float32)
    yp_ref[0:N, :] = xw * dinv


def _tc_mid_body(p_ref, yp_ref, dinv_ref, b_ref, w_ref, out_ref):
    agg = p_ref[0, 0:N, :] + p_ref[1, 0:N, :] + yp_ref[0:N, :]
    h = jnp.maximum(dinv_ref[...] * agg + b_ref[...], 0.0)
    y2 = jnp.dot(h, w_ref[...], preferred_element_type=jnp.float32, precision=lax.Precision.HIGHEST)
    out_ref[...] = jnp.zeros((NPAD, DH), jnp.float32)
    out_ref[0:N, :] = y2 * dinv_ref[...]


def _tc_final_body(q_ref, yp_ref, dinv_ref, b_ref, batch_ref, wr_ref, br_ref,
                   out_ref):
    agg = q_ref[0, 0:N, :] + q_ref[1, 0:N, :] + yp_ref[0:N, :]
    h = jnp.maximum(dinv_ref[...] * agg + b_ref[...], 0.0)
    gids = lax.broadcasted_iota(jnp.int32, (G, N), 0)
    onehot_t = (batch_ref[...] == gids).astype(jnp.float32)
    pool = lax.dot_general(onehot_t, h, (((1,), (0,)), ((), ())),
                           preferred_element_type=jnp.float32, precision=lax.Precision.HIGHEST)
    cntg = lax.dot_general(onehot_t, jnp.ones((N, 1), jnp.float32),
                           (((1,), (0,)), ((), ())),
                           preferred_element_type=jnp.float32, precision=lax.Precision.HIGHEST)
    mean = pool / jnp.maximum(cntg, 1.0)
    out_ref[...] = jnp.dot(mean, wr_ref[...],
                           preferred_element_type=jnp.float32, precision=lax.Precision.HIGHEST) + br_ref[...]


_tc_scale = pl.pallas_call(
    _tc_scale_body,
    out_shape=(
        jax.ShapeDtypeStruct((NPAD, DH), jnp.float32),
        jax.ShapeDtypeStruct((N, 1), jnp.float32),
    ),
)

_tc_mid = pl.pallas_call(
    _tc_mid_body,
    out_shape=jax.ShapeDtypeStruct((NPAD, DH), jnp.float32),
)

_tc_final = pl.pallas_call(
    _tc_final_body,
    out_shape=jax.ShapeDtypeStruct((G, DOUT), jnp.float32),
)


def kernel(x, edge_index, batch, W1, b1, W2, b2, Wr, br):
    src = edge_index[0]
    dst = edge_index[1]
    pad = jnp.full((EP - E,), N, jnp.int32)
    srcp = jnp.concatenate([src, pad]).reshape(NW, K, C)
    dstp = jnp.concatenate([dst, pad]).reshape(NW, K, C)
    zeros = jnp.zeros((NPAD, DH), jnp.float32)
    zeros_c = jnp.zeros((NPAD, CW), jnp.float32)
    ones_c = jnp.ones((C, CW), jnp.float32)
    batch_row = batch.reshape(1, N)
    b1r = b1.reshape(1, DH)
    b2r = b2.reshape(1, DH)
    brr = br.reshape(1, DOUT)

    cnt = _sc_count(dstp, ones_c, zeros_c)
    y1p, dinv = _tc_scale(cnt, x, W1)
    p = _sc_layer(y1p, srcp, dstp, zeros)
    y2p = _tc_mid(p, y1p, dinv, b1r, W2)
    q = _sc_layer(y2p, srcp, dstp, zeros)
    return _tc_final(q, y2p, dinv, b2r, batch_row, Wr, brr)


# speedup vs baseline: 28.5432x; 1.0124x over previous
import functools

import jax
import jax.numpy as jnp
from jax import lax
from jax.experimental import pallas as pl
from jax.experimental.pallas import tpu as pltpu
from jax.experimental.pallas import tpu_sc as plsc

N = 10000
E = 320000
DIN = 128
DH = 32
DOUT = 32
G = 128

NC = 2
NS = 16
NW = NC * NS
C = 128
K = -(-E // (NW * C))
K += K % 2
K2 = K // 2
EP = NW * K * C
NPAD = 10240
NT = NPAD // NS
CW = 16

_sc_mesh = plsc.VectorSubcoreMesh(
    core_axis_name="c", subcore_axis_name="s", num_cores=NC, num_subcores=NS
)


@functools.partial(
    pl.kernel,
    out_type=jax.ShapeDtypeStruct((NC, NPAD, CW), jnp.float32),
    mesh=_sc_mesh,
    scratch_types=[
        pltpu.VMEM_SHARED((NPAD, CW), jnp.float32),
        pltpu.VMEM((NT, CW), jnp.float32),
        pltpu.VMEM((K, C), jnp.int32),
        pltpu.VMEM((C, CW), jnp.float32),
    ],
    compiler_params=pltpu.CompilerParams(use_tc_tiling_on_sc=False),
)
def _sc_count(dst_hbm, ones_hbm, zero_hbm, out_hbm, accum_sh, stage_v, idx_v,
              ones_v):
    cid = lax.axis_index("c")
    sid = lax.axis_index("s")
    wid = sid * NC + cid
    rows = pl.ds(sid * NT, NT)
    pltpu.sync_copy(zero_hbm.at[rows], stage_v)
    pltpu.sync_copy(stage_v, accum_sh.at[rows])
    pltpu.sync_copy(ones_hbm, ones_v)
    pltpu.sync_copy(dst_hbm.at[wid], idx_v)
    plsc.subcore_barrier()

    def chunk(k, carry):
        pltpu.sync_copy(ones_v, accum_sh.at[idx_v.at[k]], add=True)
        return carry

    lax.fori_loop(0, K, chunk, 0)
    plsc.subcore_barrier()
    pltpu.sync_copy(accum_sh.at[rows], stage_v)
    pltpu.sync_copy(stage_v, out_hbm.at[cid, rows])


@functools.partial(
    pl.kernel,
    out_type=jax.ShapeDtypeStruct((NC, NPAD, DH), jnp.float32),
    mesh=_sc_mesh,
    scratch_types=[
        pltpu.VMEM_SHARED((NPAD, DH), jnp.float32),
        pltpu.VMEM((NT, DH), jnp.float32),
        pltpu.VMEM((K, C), jnp.int32),
        pltpu.VMEM((K, C), jnp.int32),
        [pltpu.VMEM((C, DH), jnp.float32)] * 4,
        [pltpu.SemaphoreType.DMA] * 4,
        [pltpu.SemaphoreType.DMA] * 4,
    ],
    compiler_params=pltpu.CompilerParams(use_tc_tiling_on_sc=False),
)
def _sc_layer(y_hbm, src_hbm, dst_hbm, zero_hbm, out_hbm, accum_sh,
              stage_v, sidx_v, didx_v, gbufs, gsems, ssems):
    cid = lax.axis_index("c")
    sid = lax.axis_index("s")
    wid = sid * NC + cid
    rows = pl.ds(sid * NT, NT)
    pltpu.sync_copy(zero_hbm.at[rows], stage_v)
    pltpu.sync_copy(stage_v, accum_sh.at[rows])
    pltpu.sync_copy(src_hbm.at[wid], sidx_v)
    pltpu.sync_copy(dst_hbm.at[wid], didx_v)
    plsc.subcore_barrier()

    def gstart(k, b):
        pltpu.async_copy(y_hbm.at[sidx_v.at[k]], gbufs[b], gsems[b])

    def gwait(k, b):
        pltpu.make_async_copy(y_hbm.at[sidx_v.at[k]], gbufs[b],
                              gsems[b]).wait()

    def sstart(k, b):
        pltpu.async_copy(gbufs[b], accum_sh.at[didx_v.at[k]], ssems[b],
                         add=True)

    def swait(k, b):
        pltpu.make_async_copy(gbufs[b], accum_sh.at[didx_v.at[k]],
                              ssems[b]).wait()

    gstart(0, 0)
    gstart(1, 1)
    gwait(0, 0)
    sstart(0, 0)
    gstart(2, 2)
    gwait(1, 1)
    sstart(1, 1)
    gstart(3, 3)

    def steady(i, carry):
        k0 = 2 + 4 * i
        for b in range(4):
            k = k0 + b
            bb = (2 + b) % 4
            gwait(k, bb)
            sstart(k, bb)
            swait(k - 2, (bb + 2) % 4)
            gstart(k + 2, (bb + 2) % 4)
        return carry

    lax.fori_loop(0, (K - 4) // 4, steady, 0)

    gwait(K - 2, (K - 2) % 4)
    sstart(K - 2, (K - 2) % 4)
    swait(K - 4, (K - 4) % 4)
    gwait(K - 1, (K - 1) % 4)
    sstart(K - 1, (K - 1) % 4)
    swait(K - 3, (K - 3) % 4)
    swait(K - 2, (K - 2) % 4)
    swait(K - 1, (K - 1) % 4)
    plsc.subcore_barrier()
    pltpu.sync_copy(accum_sh.at[rows], stage_v)
    pltpu.sync_copy(stage_v, out_hbm.at[cid, rows])


def _tc_scale_body(cnt_ref, x_ref, w_ref, yp_ref, dinv_ref):
    cnt = cnt_ref[0, 0:N, 0:1] + cnt_ref[1, 0:N, 0:1]
    dinv = lax.rsqrt(cnt + 1.0)
    dinv_ref[...] = dinv
    xw = jnp.dot(x_ref[...], w_ref[...], preferred_element_type=jnp.float32,
                 precision=lax.Precision.HIGHEST)
    yp_ref[...] = jnp.zeros((NPAD, DH), jnp.float32)
    yp_ref[0:N, :] = xw * dinv


def _tc_mid_body(p_ref, yp_ref, dinv_ref, b_ref, w_ref, out_ref):
    agg = p_ref[0, 0:N, :] + p_ref[1, 0:N, :] + yp_ref[0:N, :]
    h = jnp.maximum(dinv_ref[...] * agg + b_ref[...], 0.0)
    y2 = jnp.dot(h, w_ref[...], preferred_element_type=jnp.float32, precision=lax.Precision.HIGHEST)
    out_ref[...] = jnp.zeros((NPAD, DH), jnp.float32)
    out_ref[0:N, :] = y2 * dinv_ref[...]


def _tc_final_body(q_ref, yp_ref, dinv_ref, b_ref, batch_ref, wr_ref, br_ref,
                   out_ref):
    agg = q_ref[0, 0:N, :] + q_ref[1, 0:N, :] + yp_ref[0:N, :]
    h = jnp.maximum(dinv_ref[...] * agg + b_ref[...], 0.0)
    gids = lax.broadcasted_iota(jnp.int32, (G, N), 0)
    onehot_t = (batch_ref[...] == gids).astype(jnp.float32)
    pool = lax.dot_general(onehot_t, h, (((1,), (0,)), ((), ())),
                           preferred_element_type=jnp.float32, precision=lax.Precision.HIGHEST)
    cntg = lax.dot_general(onehot_t, jnp.ones((N, 1), jnp.float32),
                           (((1,), (0,)), ((), ())),
                           preferred_element_type=jnp.float32, precision=lax.Precision.HIGHEST)
    mean = pool / jnp.maximum(cntg, 1.0)
    out_ref[...] = jnp.dot(mean, wr_ref[...],
                           preferred_element_type=jnp.float32, precision=lax.Precision.HIGHEST) + br_ref[...]


_tc_scale = pl.pallas_call(
    _tc_scale_body,
    out_shape=(
        jax.ShapeDtypeStruct((NPAD, DH), jnp.float32),
        jax.ShapeDtypeStruct((N, 1), jnp.float32),
    ),
)

_tc_mid = pl.pallas_call(
    _tc_mid_body,
    out_shape=jax.ShapeDtypeStruct((NPAD, DH), jnp.float32),
)

_tc_final = pl.pallas_call(
    _tc_final_body,
    out_shape=jax.ShapeDtypeStruct((G, DOUT), jnp.float32),
)


def kernel(x, edge_index, batch, W1, b1, W2, b2, Wr, br):
    src = edge_index[0]
    dst = edge_index[1]
    pad = jnp.full((EP - E,), N, jnp.int32)
    srcp = jnp.concatenate([src, pad]).reshape(NW, K, C)
    dstp = jnp.concatenate([dst, pad]).reshape(NW, K, C)
    zeros = jnp.zeros((NPAD, DH), jnp.float32)
    zeros_c = jnp.zeros((NPAD, CW), jnp.float32)
    ones_c = jnp.ones((C, CW), jnp.float32)
    batch_row = batch.reshape(1, N)
    b1r = b1.reshape(1, DH)
    b2r = b2.reshape(1, DH)
    brr = br.reshape(1, DOUT)

    cnt = _sc_count(dstp, ones_c, zeros_c)
    y1p, dinv = _tc_scale(cnt, x, W1)
    p = _sc_layer(y1p, srcp, dstp, zeros)
    y2p = _tc_mid(p, y1p, dinv, b1r, W2)
    q = _sc_layer(y2p, srcp, dstp, zeros)
    return _tc_final(q, y2p, dinv, b2r, batch_row, Wr, brr)


# speedup vs baseline: 46.2070x; 1.6188x over previous
import functools

import jax
import jax.numpy as jnp
from jax import lax
from jax.experimental import pallas as pl
from jax.experimental.pallas import tpu as pltpu
from jax.experimental.pallas import tpu_sc as plsc

N = 10000
E = 320000
DIN = 128
DH = 32
DOUT = 32
G = 128

NC = 2
NS = 16
NW = NC * NS
C = 128
K = -(-E // (NW * C))
K += K % 2
K2 = K // 2
EP = NW * K * C
NPAD = 10240
NT = NPAD // NS
CW = 16

_sc_mesh = plsc.VectorSubcoreMesh(
    core_axis_name="c", subcore_axis_name="s", num_cores=NC, num_subcores=NS
)


@functools.partial(
    pl.kernel,
    out_type=jax.ShapeDtypeStruct((NC, NPAD, CW), jnp.float32),
    mesh=_sc_mesh,
    scratch_types=[
        pltpu.VMEM_SHARED((NPAD, CW), jnp.float32),
        pltpu.VMEM((NT, CW), jnp.float32),
        pltpu.VMEM((K, C), jnp.int32),
        pltpu.VMEM((C, CW), jnp.float32),
    ],
    compiler_params=pltpu.CompilerParams(use_tc_tiling_on_sc=False),
)
def _sc_count(dst_hbm, ones_hbm, zero_hbm, out_hbm, accum_sh, stage_v, idx_v,
              ones_v):
    cid = lax.axis_index("c")
    sid = lax.axis_index("s")
    wid = sid * NC + cid
    rows = pl.ds(sid * NT, NT)
    pltpu.sync_copy(zero_hbm.at[rows], stage_v)
    pltpu.sync_copy(stage_v, accum_sh.at[rows])
    pltpu.sync_copy(ones_hbm, ones_v)
    pltpu.sync_copy(dst_hbm.at[wid], idx_v)
    plsc.subcore_barrier()

    def chunk(k, carry):
        pltpu.sync_copy(ones_v, accum_sh.at[idx_v.at[k]], add=True)
        return carry

    lax.fori_loop(0, K, chunk, 0)
    plsc.subcore_barrier()
    pltpu.sync_copy(accum_sh.at[rows], stage_v)
    pltpu.sync_copy(stage_v, out_hbm.at[cid, rows])


@functools.partial(
    pl.kernel,
    out_type=jax.ShapeDtypeStruct((NC, NPAD, DH), jnp.float32),
    mesh=_sc_mesh,
    scratch_types=[
        pltpu.VMEM_SHARED((NPAD, DH), jnp.float32),
        pltpu.VMEM_SHARED((NPAD, DH), jnp.float32),
        pltpu.VMEM((NT, DH), jnp.float32),
        pltpu.VMEM((K, C), jnp.int32),
        pltpu.VMEM((K, C), jnp.int32),
        [pltpu.VMEM((C, DH), jnp.float32)] * 4,
        [pltpu.SemaphoreType.DMA] * 4,
        [pltpu.SemaphoreType.DMA] * 4,
    ],
    compiler_params=pltpu.CompilerParams(use_tc_tiling_on_sc=False),
)
def _sc_layer(y_hbm, src_hbm, dst_hbm, zero_hbm, out_hbm, table_sh, accum_sh,
              stage_v, sidx_v, didx_v, gbufs, gsems, ssems):
    cid = lax.axis_index("c")
    sid = lax.axis_index("s")
    wid = sid * NC + cid
    rows = pl.ds(sid * NT, NT)
    pltpu.sync_copy(y_hbm.at[rows], stage_v)
    pltpu.sync_copy(stage_v, table_sh.at[rows])
    pltpu.sync_copy(zero_hbm.at[rows], stage_v)
    pltpu.sync_copy(stage_v, accum_sh.at[rows])
    pltpu.sync_copy(src_hbm.at[wid], sidx_v)
    pltpu.sync_copy(dst_hbm.at[wid], didx_v)
    plsc.subcore_barrier()

    def gstart(k, b):
        pltpu.async_copy(table_sh.at[sidx_v.at[k]], gbufs[b], gsems[b])

    def gwait(k, b):
        pltpu.make_async_copy(table_sh.at[sidx_v.at[k]], gbufs[b],
                              gsems[b]).wait()

    def sstart(k, b):
        pltpu.async_copy(gbufs[b], accum_sh.at[didx_v.at[k]], ssems[b],
                         add=True)

    def swait(k, b):
        pltpu.make_async_copy(gbufs[b], accum_sh.at[didx_v.at[k]],
                              ssems[b]).wait()

    gstart(0, 0)
    gstart(1, 1)
    gwait(0, 0)
    sstart(0, 0)
    gstart(2, 2)
    gwait(1, 1)
    sstart(1, 1)
    gstart(3, 3)

    def steady(i, carry):
        k0 = 2 + 4 * i
        for b in range(4):
            k = k0 + b
            bb = (2 + b) % 4
            gwait(k, bb)
            sstart(k, bb)
            swait(k - 2, (bb + 2) % 4)
            gstart(k + 2, (bb + 2) % 4)
        return carry

    lax.fori_loop(0, (K - 4) // 4, steady, 0)

    gwait(K - 2, (K - 2) % 4)
    sstart(K - 2, (K - 2) % 4)
    swait(K - 4, (K - 4) % 4)
    gwait(K - 1, (K - 1) % 4)
    sstart(K - 1, (K - 1) % 4)
    swait(K - 3, (K - 3) % 4)
    swait(K - 2, (K - 2) % 4)
    swait(K - 1, (K - 1) % 4)
    plsc.subcore_barrier()
    pltpu.sync_copy(accum_sh.at[rows], stage_v)
    pltpu.sync_copy(stage_v, out_hbm.at[cid, rows])


def _tc_scale_body(cnt_ref, x_ref, w_ref, yp_ref, dinv_ref):
    cnt = cnt_ref[0, 0:N, 0:1] + cnt_ref[1, 0:N, 0:1]
    dinv = lax.rsqrt(cnt + 1.0)
    dinv_ref[...] = dinv
    xw = jnp.dot(x_ref[...], w_ref[...], preferred_element_type=jnp.float32,
                 precision=lax.Precision.HIGHEST)
    yp_ref[...] = jnp.zeros((NPAD, DH), jnp.float32)
    yp_ref[0:N, :] = xw * dinv


def _tc_mid_body(p_ref, yp_ref, dinv_ref, b_ref, w_ref, out_ref):
    agg = p_ref[0, 0:N, :] + p_ref[1, 0:N, :] + yp_ref[0:N, :]
    h = jnp.maximum(dinv_ref[...] * agg + b_ref[...], 0.0)
    y2 = jnp.dot(h, w_ref[...], preferred_element_type=jnp.float32, precision=lax.Precision.HIGHEST)
    out_ref[...] = jnp.zeros((NPAD, DH), jnp.float32)
    out_ref[0:N, :] = y2 * dinv_ref[...]


def _tc_final_body(q_ref, yp_ref, dinv_ref, b_ref, batch_ref, wr_ref, br_ref,
                   out_ref):
    agg = q_ref[0, 0:N, :] + q_ref[1, 0:N, :] + yp_ref[0:N, :]
    h = jnp.maximum(dinv_ref[...] * agg + b_ref[...], 0.0)
    gids = lax.broadcasted_iota(jnp.int32, (G, N), 0)
    onehot_t = (batch_ref[...] == gids).astype(jnp.float32)
    pool = lax.dot_general(onehot_t, h, (((1,), (0,)), ((), ())),
                           preferred_element_type=jnp.float32, precision=lax.Precision.HIGHEST)
    cntg = lax.dot_general(onehot_t, jnp.ones((N, 1), jnp.float32),
                           (((1,), (0,)), ((), ())),
                           preferred_element_type=jnp.float32, precision=lax.Precision.HIGHEST)
    mean = pool / jnp.maximum(cntg, 1.0)
    out_ref[...] = jnp.dot(mean, wr_ref[...],
                           preferred_element_type=jnp.float32, precision=lax.Precision.HIGHEST) + br_ref[...]


_tc_scale = pl.pallas_call(
    _tc_scale_body,
    out_shape=(
        jax.ShapeDtypeStruct((NPAD, DH), jnp.float32),
        jax.ShapeDtypeStruct((N, 1), jnp.float32),
    ),
)

_tc_mid = pl.pallas_call(
    _tc_mid_body,
    out_shape=jax.ShapeDtypeStruct((NPAD, DH), jnp.float32),
)

_tc_final = pl.pallas_call(
    _tc_final_body,
    out_shape=jax.ShapeDtypeStruct((G, DOUT), jnp.float32),
)


def kernel(x, edge_index, batch, W1, b1, W2, b2, Wr, br):
    src = edge_index[0]
    dst = edge_index[1]
    pad = jnp.full((EP - E,), N, jnp.int32)
    srcp = jnp.concatenate([src, pad]).reshape(NW, K, C)
    dstp = jnp.concatenate([dst, pad]).reshape(NW, K, C)
    zeros = jnp.zeros((NPAD, DH), jnp.float32)
    zeros_c = jnp.zeros((NPAD, CW), jnp.float32)
    ones_c = jnp.ones((C, CW), jnp.float32)
    batch_row = batch.reshape(1, N)
    b1r = b1.reshape(1, DH)
    b2r = b2.reshape(1, DH)
    brr = br.reshape(1, DOUT)

    cnt = _sc_count(dstp, ones_c, zeros_c)
    y1p, dinv = _tc_scale(cnt, x, W1)
    p = _sc_layer(y1p, srcp, dstp, zeros)
    y2p = _tc_mid(p, y1p, dinv, b1r, W2)
    q = _sc_layer(y2p, srcp, dstp, zeros)
    return _tc_final(q, y2p, dinv, b2r, batch_row, Wr, brr)


# speedup vs baseline: 57.1476x; 1.2368x over previous
import functools

import jax
import jax.numpy as jnp
from jax import lax
from jax.experimental import pallas as pl
from jax.experimental.pallas import tpu as pltpu
from jax.experimental.pallas import tpu_sc as plsc

N = 10000
E = 320000
DIN = 128
DH = 32
DOUT = 32
G = 128

NC = 2
NS = 16
NW = NC * NS
C = 128
K = -(-E // (NW * C))
K += K % 2
EP = NW * K * C
NPAD = 10240
NT = NPAD // NS
CW = 32
NR = NPAD * DH // 128
NG = N * DH // 128

_sc_mesh = plsc.VectorSubcoreMesh(
    core_axis_name="c", subcore_axis_name="s", num_cores=NC, num_subcores=NS
)


@functools.partial(
    pl.kernel,
    out_type=jax.ShapeDtypeStruct((NC, NPAD, CW), jnp.float32),
    mesh=_sc_mesh,
    scratch_types=[
        pltpu.VMEM_SHARED((NPAD, CW), jnp.float32),
        pltpu.VMEM((NT, CW), jnp.float32),
        pltpu.VMEM((K, C), jnp.int32),
        pltpu.VMEM((C, CW), jnp.float32),
        pltpu.SemaphoreType.DMA,
    ],
    compiler_params=pltpu.CompilerParams(use_tc_tiling_on_sc=False),
)
def _sc_count(dst_hbm, ones_hbm, zero_hbm, out_hbm, accum_sh, stage_v, idx_v,
              ones_v, sem):
    cid = lax.axis_index("c")
    sid = lax.axis_index("s")
    wid = sid * NC + cid
    rows = pl.ds(sid * NT, NT)
    pltpu.sync_copy(zero_hbm.at[rows], stage_v)
    pltpu.sync_copy(stage_v, accum_sh.at[rows])
    pltpu.sync_copy(ones_hbm, ones_v)
    pltpu.sync_copy(dst_hbm.at[wid], idx_v)
    plsc.subcore_barrier()

    def chunk(k, carry):
        pltpu.async_copy(ones_v, accum_sh.at[idx_v.at[k]], sem, add=True)
        return carry

    lax.fori_loop(0, K, chunk, 0)

    def drain(k, carry):
        pltpu.make_async_copy(ones_v, accum_sh.at[idx_v.at[k]], sem).wait()
        return carry

    lax.fori_loop(0, K, drain, 0)
    plsc.subcore_barrier()
    pltpu.sync_copy(accum_sh.at[rows], stage_v)
    pltpu.sync_copy(stage_v, out_hbm.at[cid, rows])


@functools.partial(
    pl.kernel,
    out_type=jax.ShapeDtypeStruct((NC, NPAD, DH), jnp.float32),
    mesh=_sc_mesh,
    scratch_types=[
        pltpu.VMEM_SHARED((NPAD, DH), jnp.float32),
        pltpu.VMEM_SHARED((NPAD, DH), jnp.float32),
        pltpu.VMEM((NT, DH), jnp.float32),
        pltpu.VMEM((K, C), jnp.int32),
        pltpu.VMEM((K, C), jnp.int32),
        [pltpu.VMEM((C, DH), jnp.float32)] * 4,
        [pltpu.SemaphoreType.DMA] * 4,
        [pltpu.SemaphoreType.DMA] * 4,
    ],
    compiler_params=pltpu.CompilerParams(use_tc_tiling_on_sc=False),
)
def _sc_layer(y_hbm, src_hbm, dst_hbm, zero_hbm, out_hbm, table_sh, accum_sh,
              stage_v, sidx_v, didx_v, gbufs, gsems, ssems):
    cid = lax.axis_index("c")
    sid = lax.axis_index("s")
    wid = sid * NC + cid
    rows = pl.ds(sid * NT, NT)
    pltpu.sync_copy(y_hbm.at[rows], stage_v)
    pltpu.sync_copy(stage_v, table_sh.at[rows])
    pltpu.sync_copy(zero_hbm.at[rows], stage_v)
    pltpu.sync_copy(stage_v, accum_sh.at[rows])
    pltpu.sync_copy(src_hbm.at[wid], sidx_v)
    pltpu.sync_copy(dst_hbm.at[wid], didx_v)
    plsc.subcore_barrier()

    def gstart(k, b):
        pltpu.async_copy(table_sh.at[sidx_v.at[k]], gbufs[b], gsems[b])

    def gwait(k, b):
        pltpu.make_async_copy(table_sh.at[sidx_v.at[k]], gbufs[b],
                              gsems[b]).wait()

    def sstart(k, b):
        pltpu.async_copy(gbufs[b], accum_sh.at[didx_v.at[k]], ssems[b],
                         add=True)

    def swait(k, b):
        pltpu.make_async_copy(gbufs[b], accum_sh.at[didx_v.at[k]],
                              ssems[b]).wait()

    gstart(0, 0)
    gstart(1, 1)
    gwait(0, 0)
    sstart(0, 0)
    gstart(2, 2)
    gwait(1, 1)
    sstart(1, 1)
    gstart(3, 3)

    def steady(i, carry):
        k0 = 2 + 4 * i
        for b in range(4):
            k = k0 + b
            bb = (2 + b) % 4
            gwait(k, bb)
            sstart(k, bb)
            swait(k - 2, (bb + 2) % 4)
            gstart(k + 2, (bb + 2) % 4)
        return carry

    lax.fori_loop(0, (K - 4) // 4, steady, 0)

    gwait(K - 2, (K - 2) % 4)
    sstart(K - 2, (K - 2) % 4)
    swait(K - 4, (K - 4) % 4)
    gwait(K - 1, (K - 1) % 4)
    sstart(K - 1, (K - 1) % 4)
    swait(K - 3, (K - 3) % 4)
    swait(K - 2, (K - 2) % 4)
    swait(K - 1, (K - 1) % 4)
    plsc.subcore_barrier()
    pltpu.sync_copy(accum_sh.at[rows], stage_v)
    pltpu.sync_copy(stage_v, out_hbm.at[cid, rows])


def _bias128(b_ref):
    b = b_ref[...]
    return jnp.concatenate([b, b, b, b], axis=1)


def _rowmask():
    return (lax.broadcasted_iota(jnp.int32, (NR, 128), 0)
            < NG).astype(jnp.float32)


def _tc_scale_body(cnt_ref, x4_ref, wblk_ref, yp_ref, dinvr_ref):
    crow = cnt_ref[0] + cnt_ref[1]
    dinvr = lax.rsqrt(crow + 1.0)
    dinvr_ref[...] = dinvr
    xw = jnp.dot(x4_ref[...], wblk_ref[...],
                 preferred_element_type=jnp.float32,
                 precision=lax.Precision.HIGHEST)
    xw_full = jnp.concatenate(
        [xw, jnp.zeros((NR - NG, 128), jnp.float32)], axis=0)
    yp_ref[...] = xw_full * dinvr


def _tc_mid_body(p_ref, yp_ref, dinvr_ref, b_ref, wblk_ref, out_ref):
    agg = p_ref[0] + p_ref[1] + yp_ref[...]
    h = jnp.maximum(dinvr_ref[...] * agg + _bias128(b_ref), 0.0)
    y2 = jnp.dot(h, wblk_ref[...], preferred_element_type=jnp.float32,
                 precision=lax.Precision.HIGHEST)
    out_ref[...] = y2 * dinvr_ref[...] * _rowmask()


def _tc_final_body(q_ref, yp_ref, dinvr_ref, b_ref, batch4_ref, wr_ref,
                   br_ref, out_ref):
    agg = q_ref[0] + q_ref[1] + yp_ref[...]
    h = jnp.maximum(dinvr_ref[...] * agg + _bias128(b_ref), 0.0)
    gids = lax.broadcasted_iota(jnp.int32, (G, NR), 0)
    pool = jnp.zeros((G, DH), jnp.float32)
    cntg = jnp.zeros((G, 1), jnp.float32)
    ones_r = jnp.ones((NR, 1), jnp.float32)
    for j in range(4):
        ohj = (batch4_ref[j:j + 1, :] == gids).astype(jnp.float32)
        pool = pool + lax.dot_general(
            ohj, h[:, 32 * j:32 * j + 32], (((1,), (0,)), ((), ())),
            preferred_element_type=jnp.float32,
            precision=lax.Precision.HIGHEST)
        cntg = cntg + lax.dot_general(
            ohj, ones_r, (((1,), (0,)), ((), ())),
            preferred_element_type=jnp.float32,
            precision=lax.Precision.HIGHEST)
    mean = pool / jnp.maximum(cntg, 1.0)
    out_ref[...] = jnp.dot(mean, wr_ref[...],
                           preferred_element_type=jnp.float32,
                           precision=lax.Precision.HIGHEST) + br_ref[...]


_tc_scale = pl.pallas_call(
    _tc_scale_body,
    out_shape=(
        jax.ShapeDtypeStruct((NR, 128), jnp.float32),
        jax.ShapeDtypeStruct((NR, 128), jnp.float32),
    ),
)

_tc_mid = pl.pallas_call(
    _tc_mid_body,
    out_shape=jax.ShapeDtypeStruct((NR, 128), jnp.float32),
)

_tc_final = pl.pallas_call(
    _tc_final_body,
    out_shape=jax.ShapeDtypeStruct((G, DOUT), jnp.float32),
)


def _blockdiag4(w):
    d_in, d_out = w.shape
    out = jnp.zeros((4 * d_in, 4 * d_out), w.dtype)
    for j in range(4):
        out = lax.dynamic_update_slice(out, w, (j * d_in, j * d_out))
    return out


def kernel(x, edge_index, batch, W1, b1, W2, b2, Wr, br):
    src = edge_index[0]
    dst = edge_index[1]
    pad = jnp.full((EP - E,), N, jnp.int32)
    srcp = jnp.concatenate([src, pad]).reshape(NW, K, C)
    dstp = jnp.concatenate([dst, pad]).reshape(NW, K, C)
    zeros = jnp.zeros((NPAD, DH), jnp.float32)
    ones_c = jnp.ones((C, CW), jnp.float32)
    x4 = x.reshape(NG, 4 * DIN)
    w1blk = _blockdiag4(W1)
    w2blk = _blockdiag4(W2)
    batch4 = jnp.concatenate(
        [batch, jnp.full((NPAD - N,), -1, jnp.int32)]).reshape(NR, 4).T
    b1r = b1.reshape(1, DH)
    b2r = b2.reshape(1, DH)
    brr = br.reshape(1, DOUT)

    cnt = _sc_count(dstp, ones_c, zeros)
    y1p128, dinvr = _tc_scale(cnt.reshape(NC, NR, 128), x4, w1blk)
    p = _sc_layer(y1p128.reshape(NPAD, DH), srcp, dstp, zeros)
    y2p128 = _tc_mid(p.reshape(NC, NR, 128), y1p128, dinvr, b1r, w2blk)
    q = _sc_layer(y2p128.reshape(NPAD, DH), srcp, dstp, zeros)
    return _tc_final(q.reshape(NC, NR, 128), y2p128, dinvr, b2r, batch4,
                     Wr, brr)
